# baseline (speedup 1.0000x reference)
import jax
import jax.numpy as jnp
from jax.experimental import pallas as pl


def _pair_kernel(x_ref, o_ref):
    Bo, F, D = x_ref.shape
    C = 128
    for c in range(Bo // C):
        xb = x_ref[c * C : (c + 1) * C]
        xt = [jnp.transpose(xb[:, i, :]) for i in range(F)]
        rows = []
        for i in range(F - 1):
            for j in range(i + 1, F):
                rows.append(jnp.sum(xt[i] * xt[j], axis=0, keepdims=True))
        pt = jnp.concatenate(rows, axis=0)
        o_ref[c * C : (c + 1) * C] = jnp.transpose(pt)


def kernel(x):
    N, F, D = x.shape
    P = F * (F - 1) // 2
    B = 512
    return pl.pallas_call(
        _pair_kernel,
        grid=(N // B,),
        in_specs=[pl.BlockSpec((B, F, D), lambda n: (n, 0, 0))],
        out_specs=pl.BlockSpec((B, P), lambda n: (n, 0)),
        out_shape=jax.ShapeDtypeStruct((N, P), x.dtype),
    )(x)


# speedup vs baseline: 1.5726x; 1.5726x over previous
import jax
import jax.numpy as jnp
from jax.experimental import pallas as pl


def _pair_kernel(x_ref, o_ref):
    Bo, F, D = x_ref.shape
    C = 128
    for c in range(Bo // C):
        xb = x_ref[c * C : (c + 1) * C]
        g = jax.lax.dot_general(xb, xb, (((2,), (2,)), ((0,), (0,))))
        bands = [g[:, i, i + 1 :] for i in range(F - 1)]
        o_ref[c * C : (c + 1) * C] = jnp.concatenate(bands, axis=-1)


def kernel(x):
    N, F, D = x.shape
    P = F * (F - 1) // 2
    B = 512
    return pl.pallas_call(
        _pair_kernel,
        grid=(N // B,),
        in_specs=[pl.BlockSpec((B, F, D), lambda n: (n, 0, 0))],
        out_specs=pl.BlockSpec((B, P), lambda n: (n, 0)),
        out_shape=jax.ShapeDtypeStruct((N, P), x.dtype),
    )(x)


# speedup vs baseline: 1.5902x; 1.0112x over previous
import jax
import jax.numpy as jnp
from jax.experimental import pallas as pl


def _pair_kernel(x_ref, o_ref):
    Bo, F, D = x_ref.shape
    C = 128
    for c in range(Bo // C):
        xb = x_ref[c * C : (c + 1) * C]
        g = jax.lax.dot_general(xb, xb, (((2,), (2,)), ((0,), (0,))))
        off = 0
        for i in range(F - 1):
            w = F - 1 - i
            o_ref[c * C : (c + 1) * C, off : off + w] = g[:, i, i + 1 :]
            off += w


def kernel(x):
    N, F, D = x.shape
    P = F * (F - 1) // 2
    B = 512
    return pl.pallas_call(
        _pair_kernel,
        grid=(N // B,),
        in_specs=[pl.BlockSpec((B, F, D), lambda n: (n, 0, 0))],
        out_specs=pl.BlockSpec((B, P), lambda n: (n, 0)),
        out_shape=jax.ShapeDtypeStruct((N, P), x.dtype),
    )(x)
